# baseline (device time: 85574 ns/iter reference)
import jax
import jax.numpy as jnp
from jax import lax
from jax.experimental import pallas as pl
from jax.experimental.pallas import tpu as pltpu

M = 4096
N = 4096
T = 512
G = M // T
RH = 8
CH = 128
S = 4
TS = T // S


def kernel(x):
    def body(x_hbm, o_hbm, buf, out_buf, row_halo, col_halo,
             load_sems, out_sems, row_send, row_recv, col_send, col_recv):
        i = pl.program_id(0)
        slot = lax.rem(i, 2)
        nslot = lax.rem(i + 1, 2)
        my_x = lax.axis_index("x")
        my_y = lax.axis_index("y")
        x_nbr = (1 - my_x, my_y)
        y_nbr = (my_x, 1 - my_y)
        send_col = (1 - my_y) * (N - CH)

        def col_chunk(k):
            return pltpu.make_async_remote_copy(
                src_ref=x_hbm.at[pl.ds(k * T, T), pl.ds(send_col, CH)],
                dst_ref=col_halo.at[pl.ds(k * T, T), :],
                send_sem=col_send.at[k],
                recv_sem=col_recv.at[k],
                device_id=y_nbr,
                device_id_type=pl.DeviceIdType.MESH,
            )

        def main_stripes(k, s):
            return [
                pltpu.make_async_copy(
                    x_hbm.at[pl.ds(k * T + j * TS, TS), :],
                    buf.at[s, pl.ds(RH + j * TS, TS), :],
                    load_sems.at[s, j],
                )
                for j in range(S)
            ]

        def top_halo_cp(src, s):
            return pltpu.make_async_copy(
                src, buf.at[s, pl.ds(0, RH), :], load_sems.at[s, S]
            )

        def bot_halo_cp(src, s):
            return pltpu.make_async_copy(
                src, buf.at[s, pl.ds(T + RH, RH), :], load_sems.at[s, S + 1]
            )

        def out_stripes(k, s):
            return [
                pltpu.make_async_copy(
                    out_buf.at[s, pl.ds(j * TS, TS), :],
                    o_hbm.at[pl.ds(k * T + j * TS, TS), :],
                    out_sems.at[s, j],
                )
                for j in range(S)
            ]

        @pl.when(i == 0)
        def _prologue():
            for c in main_stripes(0, 0):
                c.start()
            bot_halo_cp(x_hbm.at[pl.ds(T, RH), :], 0).start()

            barrier_sem = pltpu.get_barrier_semaphore()
            for nbr in (x_nbr, y_nbr):
                pl.semaphore_signal(
                    barrier_sem, inc=1,
                    device_id=nbr, device_id_type=pl.DeviceIdType.MESH,
                )
            pl.semaphore_wait(barrier_sem, 2)

            send_row = (1 - my_x) * (M - RH)
            row_rdma = pltpu.make_async_remote_copy(
                src_ref=x_hbm.at[pl.ds(send_row, RH), :],
                dst_ref=row_halo,
                send_sem=row_send,
                recv_sem=row_recv,
                device_id=x_nbr,
                device_id_type=pl.DeviceIdType.MESH,
            )
            row_rdma.start()
            for k in range(G):
                col_chunk(k).start()

            row_rdma.wait()
            top_halo_cp(row_halo, 0).start()

        @pl.when(i < G - 1)
        def _prefetch():
            k0 = (i + 1) * T
            for c in main_stripes(i + 1, nslot):
                c.start()
            top_halo_cp(x_hbm.at[pl.ds(k0 - RH, RH), :], nslot).start()

            @pl.when(i + 1 < G - 1)
            def _bot_local():
                bot_halo_cp(x_hbm.at[pl.ds(k0 + T, RH), :], nslot).start()

            @pl.when(i + 1 == G - 1)
            def _bot_halo():
                bot_halo_cp(row_halo, nslot).start()

        for c in main_stripes(i, slot):
            c.wait()
        top_halo_cp(row_halo, slot).wait()
        bot_halo_cp(row_halo, slot).wait()
        col_chunk(i).wait()

        @pl.when(i >= 2)
        def _free_out():
            for c in out_stripes(i - 2, slot):
                c.wait()

        center = buf[slot, RH:T + RH, :]
        north = buf[slot, RH - 1:T + RH - 1, :]
        south = buf[slot, RH + 1:T + RH + 1, :]
        r0 = i * T
        hblk = col_halo[pl.ds(r0, T), :]
        hcol = jnp.where(my_y == 1, hblk[:, CH - 1:CH], hblk[:, 0:1])
        west = jnp.concatenate([hcol, center[:, :N - 1]], axis=1)
        east = jnp.concatenate([center[:, 1:], hcol], axis=1)

        stencil = 0.5 * center + 0.125 * (north + south + west + east)
        out_buf[slot, :, :] = stencil.astype(jnp.bfloat16)

        @pl.when(my_y == 0)
        def _west_edge():
            out_buf[slot, :, 0:1] = center[:, 0:1].astype(jnp.bfloat16)

        @pl.when(my_y == 1)
        def _east_edge():
            out_buf[slot, :, N - 1:N] = center[:, N - 1:N].astype(jnp.bfloat16)

        @pl.when((i == 0) & (my_x == 0))
        def _north_edge():
            out_buf[slot, 0:1, :] = center[0:1, :].astype(jnp.bfloat16)

        @pl.when((i == G - 1) & (my_x == 1))
        def _south_edge():
            out_buf[slot, T - 1:T, :] = center[T - 1:T, :].astype(jnp.bfloat16)

        for c in out_stripes(i, slot):
            c.start()

        @pl.when(i == G - 1)
        def _drain():
            for c in out_stripes(G - 2, nslot):
                c.wait()
            for c in out_stripes(G - 1, slot):
                c.wait()

    return pl.pallas_call(
        body,
        grid=(G,),
        out_shape=jax.ShapeDtypeStruct((M, N), jnp.bfloat16),
        in_specs=[pl.BlockSpec(memory_space=pl.ANY)],
        out_specs=pl.BlockSpec(memory_space=pl.ANY),
        scratch_shapes=[
            pltpu.VMEM((2, T + 2 * RH, N), jnp.float32),
            pltpu.VMEM((2, T, N), jnp.bfloat16),
            pltpu.VMEM((RH, N), jnp.float32),
            pltpu.VMEM((M, CH), jnp.float32),
            pltpu.SemaphoreType.DMA((2, S + 2)),
            pltpu.SemaphoreType.DMA((2, S)),
            pltpu.SemaphoreType.DMA,
            pltpu.SemaphoreType.DMA,
            pltpu.SemaphoreType.DMA((G,)),
            pltpu.SemaphoreType.DMA((G,)),
        ],
        compiler_params=pltpu.CompilerParams(
            collective_id=0,
            dimension_semantics=("arbitrary",),
            vmem_limit_bytes=64 * 1024 * 1024,
        ),
    )(x)


# device time: 85094 ns/iter; 1.0056x vs baseline; 1.0056x over previous
import jax
import jax.numpy as jnp
from jax import lax
from jax.experimental import pallas as pl
from jax.experimental.pallas import tpu as pltpu

M = 4096
N = 4096
T = 512
G = M // T
RH = 8
CH = 128
S = 2
TS = T // S


def kernel(x):
    def body(x_hbm, o_hbm, buf, out_buf, row_halo, col_halo,
             load_sems, out_sems, row_send, row_recv, col_send, col_recv):
        i = pl.program_id(0)
        slot = lax.rem(i, 2)
        nslot = lax.rem(i + 1, 2)
        my_x = lax.axis_index("x")
        my_y = lax.axis_index("y")
        x_nbr = (1 - my_x, my_y)
        y_nbr = (my_x, 1 - my_y)
        send_col = (1 - my_y) * (N - CH)

        def col_chunk(k):
            return pltpu.make_async_remote_copy(
                src_ref=x_hbm.at[pl.ds(k * T, T), pl.ds(send_col, CH)],
                dst_ref=col_halo.at[pl.ds(k * T, T), :],
                send_sem=col_send.at[k],
                recv_sem=col_recv.at[k],
                device_id=y_nbr,
                device_id_type=pl.DeviceIdType.MESH,
            )

        def main_stripes(k, s):
            return [
                pltpu.make_async_copy(
                    x_hbm.at[pl.ds(k * T + j * TS, TS), :],
                    buf.at[s, pl.ds(RH + j * TS, TS), :],
                    load_sems.at[s, j],
                )
                for j in range(S)
            ]

        def top_halo_cp(src, s):
            return pltpu.make_async_copy(
                src, buf.at[s, pl.ds(0, RH), :], load_sems.at[s, S]
            )

        def bot_halo_cp(src, s):
            return pltpu.make_async_copy(
                src, buf.at[s, pl.ds(T + RH, RH), :], load_sems.at[s, S + 1]
            )

        def out_stripes(k, s):
            return [
                pltpu.make_async_copy(
                    out_buf.at[s, pl.ds(j * TS, TS), :],
                    o_hbm.at[pl.ds(k * T + j * TS, TS), :],
                    out_sems.at[s, j],
                )
                for j in range(S)
            ]

        @pl.when(i == 0)
        def _prologue():
            for c in main_stripes(0, 0):
                c.start()
            bot_halo_cp(x_hbm.at[pl.ds(T, RH), :], 0).start()

            barrier_sem = pltpu.get_barrier_semaphore()
            for nbr in (x_nbr, y_nbr):
                pl.semaphore_signal(
                    barrier_sem, inc=1,
                    device_id=nbr, device_id_type=pl.DeviceIdType.MESH,
                )
            pl.semaphore_wait(barrier_sem, 2)

            send_row = (1 - my_x) * (M - RH)
            row_rdma = pltpu.make_async_remote_copy(
                src_ref=x_hbm.at[pl.ds(send_row, RH), :],
                dst_ref=row_halo,
                send_sem=row_send,
                recv_sem=row_recv,
                device_id=x_nbr,
                device_id_type=pl.DeviceIdType.MESH,
            )
            row_rdma.start()
            for k in range(G):
                col_chunk(k).start()

            row_rdma.wait()
            top_halo_cp(row_halo, 0).start()

        @pl.when(i < G - 1)
        def _prefetch():
            k0 = (i + 1) * T
            for c in main_stripes(i + 1, nslot):
                c.start()
            top_halo_cp(x_hbm.at[pl.ds(k0 - RH, RH), :], nslot).start()

            @pl.when(i + 1 < G - 1)
            def _bot_local():
                bot_halo_cp(x_hbm.at[pl.ds(k0 + T, RH), :], nslot).start()

            @pl.when(i + 1 == G - 1)
            def _bot_halo():
                bot_halo_cp(row_halo, nslot).start()

        for c in main_stripes(i, slot):
            c.wait()
        top_halo_cp(row_halo, slot).wait()
        bot_halo_cp(row_halo, slot).wait()
        col_chunk(i).wait()

        @pl.when(i >= 2)
        def _free_out():
            for c in out_stripes(i - 2, slot):
                c.wait()

        center = buf[slot, RH:T + RH, :]
        north = buf[slot, RH - 1:T + RH - 1, :]
        south = buf[slot, RH + 1:T + RH + 1, :]
        r0 = i * T
        hblk = col_halo[pl.ds(r0, T), :]
        hcol = jnp.where(my_y == 1, hblk[:, CH - 1:CH], hblk[:, 0:1])
        west = jnp.concatenate([hcol, center[:, :N - 1]], axis=1)
        east = jnp.concatenate([center[:, 1:], hcol], axis=1)

        stencil = 0.5 * center + 0.125 * (north + south + west + east)
        out_buf[slot, :, :] = stencil.astype(jnp.bfloat16)

        @pl.when(my_y == 0)
        def _west_edge():
            out_buf[slot, :, 0:1] = center[:, 0:1].astype(jnp.bfloat16)

        @pl.when(my_y == 1)
        def _east_edge():
            out_buf[slot, :, N - 1:N] = center[:, N - 1:N].astype(jnp.bfloat16)

        @pl.when((i == 0) & (my_x == 0))
        def _north_edge():
            out_buf[slot, 0:1, :] = center[0:1, :].astype(jnp.bfloat16)

        @pl.when((i == G - 1) & (my_x == 1))
        def _south_edge():
            out_buf[slot, T - 1:T, :] = center[T - 1:T, :].astype(jnp.bfloat16)

        for c in out_stripes(i, slot):
            c.start()

        @pl.when(i == G - 1)
        def _drain():
            for c in out_stripes(G - 2, nslot):
                c.wait()
            for c in out_stripes(G - 1, slot):
                c.wait()

    return pl.pallas_call(
        body,
        grid=(G,),
        out_shape=jax.ShapeDtypeStruct((M, N), jnp.bfloat16),
        in_specs=[pl.BlockSpec(memory_space=pl.ANY)],
        out_specs=pl.BlockSpec(memory_space=pl.ANY),
        scratch_shapes=[
            pltpu.VMEM((2, T + 2 * RH, N), jnp.float32),
            pltpu.VMEM((2, T, N), jnp.bfloat16),
            pltpu.VMEM((RH, N), jnp.float32),
            pltpu.VMEM((M, CH), jnp.float32),
            pltpu.SemaphoreType.DMA((2, S + 2)),
            pltpu.SemaphoreType.DMA((2, S)),
            pltpu.SemaphoreType.DMA,
            pltpu.SemaphoreType.DMA,
            pltpu.SemaphoreType.DMA((G,)),
            pltpu.SemaphoreType.DMA((G,)),
        ],
        compiler_params=pltpu.CompilerParams(
            collective_id=0,
            dimension_semantics=("arbitrary",),
            vmem_limit_bytes=64 * 1024 * 1024,
        ),
    )(x)
